# baseline (speedup 1.0000x reference)
import functools

import jax
import jax.numpy as jnp
from jax import lax
from jax.experimental import pallas as pl
from jax.experimental.pallas import tpu as pltpu
from jax.experimental.pallas import tpu_sc as plsc

N = 8192
D_MODEL = 2048
D_SAE = 16384
K = 64

EM = 512
EN = 1024
CHUNK = 256
DM = 512
DK = 2048

NWORK = 32
ROWS_PER_W = N // NWORK
NVREG = D_SAE // 16
SURV_CAP = 2048


def _encode_kernel(x_ref, w_ref, b_ref, z_ref, c_ref, cmin_sc):
    i = pl.program_id(1)
    z = lax.dot_general(
        x_ref[...].astype(jnp.bfloat16), w_ref[...].astype(jnp.bfloat16),
        (((1,), (1,)), ((), ())),
        preferred_element_type=jnp.float32,
    )
    z = z + b_ref[...]
    z_ref[...] = z
    cm = jnp.max(z[:, 0:CHUNK], axis=1, keepdims=True)
    for k in range(1, EN // CHUNK):
        cm = jnp.minimum(cm, jnp.max(z[:, k * CHUNK:(k + 1) * CHUNK],
                                     axis=1, keepdims=True))

    @pl.when(i == 0)
    def _():
        cmin_sc[...] = cm

    @pl.when(i != 0)
    def _():
        cmin_sc[...] = jnp.minimum(cmin_sc[...], cm)

    @pl.when(i == pl.num_programs(1) - 1)
    def _():
        c_ref[...] = jnp.minimum(cmin_sc[...], cm)


def _encode(x, W_enc_bf, b_enc):
    nr = x.shape[0]
    m_blocks = nr // EM
    n_blocks = D_SAE // EN
    return pl.pallas_call(
        _encode_kernel,
        grid=(m_blocks, n_blocks),
        in_specs=[
            pl.BlockSpec((EM, D_MODEL), lambda j, i: (j, 0)),
            pl.BlockSpec((EN, D_MODEL), lambda j, i: (i, 0)),
            pl.BlockSpec((1, EN), lambda j, i: (0, i)),
        ],
        out_specs=[
            pl.BlockSpec((EM, EN), lambda j, i: (j, i)),
            pl.BlockSpec((EM, 1), lambda j, i: (j, 0)),
        ],
        out_shape=[
            jax.ShapeDtypeStruct((nr, D_SAE), jnp.float32),
            jax.ShapeDtypeStruct((nr, 1), jnp.float32),
        ],
        scratch_shapes=[pltpu.VMEM((EM, 1), jnp.float32)],
    )(x, W_enc_bf, b_enc.reshape(1, D_SAE))


def _monotone_key(v):
    u = plsc.bitcast(v, jnp.int32)
    s = lax.shift_right_arithmetic(u, 31)
    m = lax.bitwise_or(s, jnp.int32(-2147483648))
    return plsc.bitcast(lax.bitwise_xor(u, m), jnp.uint32)


def _inv_monotone_key(k):
    ki = plsc.bitcast(k, jnp.int32)
    nki = lax.bitwise_xor(ki, jnp.int32(-1))
    m = lax.bitwise_or(
        lax.shift_right_arithmetic(nki, 31),
        jnp.int32(-2147483648))
    return plsc.bitcast(lax.bitwise_xor(ki, m), jnp.float32)


def _sc_compact_body(rpw, z_hbm, c_hbm, keys_hbm, ns_hbm,
                     row0, row1, sv0, sv1, c_v, ns_v,
                     s_in0, s_in1, s_o0, s_o1):
    wid = lax.axis_index("s") * 2 + lax.axis_index("c")
    base_row = wid * rpw

    lane = lax.iota(jnp.int32, 16)
    zero16i = jnp.zeros((16,), jnp.int32)
    cap16 = jnp.full((16,), SURV_CAP - 16, jnp.int32)

    pltpu.sync_copy(c_hbm.at[pl.ds(base_row, rpw)], c_v)

    def process_row(rr, row_v, sval):
        cchunk = c_v[pl.ds((rr // 16) * 16, 16)]
        csel = jnp.where(lane == (rr % 16).astype(jnp.int32), cchunk,
                         jnp.float32(-jnp.inf))
        c_b = jnp.full((16,), jnp.max(csel), jnp.float32)

        def compact_body(i, off):
            v = row_v[pl.ds(i * 16, 16)]
            mask = v >= c_b
            mi = mask.astype(jnp.int32)
            dest = off + plsc.cumsum(mi) - mi
            plsc.store_scatter(sval, [dest], v, mask=mask)
            pc = plsc.all_reduce_population_count(mask)
            return jnp.minimum(off + pc, cap16)
        off_vec = lax.fori_loop(0, NVREG, compact_body, zero16i)
        ns_v[pl.ds(rr * 16, 16)] = off_vec
        return 0

    PAIRS = rpw // 2
    pltpu.make_async_copy(z_hbm.at[base_row], row0, s_in0).start()

    def pair_body(i, _):
        r = base_row + 2 * i
        pltpu.make_async_copy(z_hbm.at[r + 1], row1, s_in1).start()
        pltpu.make_async_copy(z_hbm.at[r], row0, s_in0).wait()

        @pl.when(i > 0)
        def _():
            pltpu.make_async_copy(sv0, keys_hbm.at[r - 2], s_o0).wait()
        process_row(2 * i, row0, sv0)
        pltpu.make_async_copy(sv0, keys_hbm.at[r], s_o0).start()

        @pl.when(i < PAIRS - 1)
        def _():
            pltpu.make_async_copy(z_hbm.at[r + 2], row0, s_in0).start()
        pltpu.make_async_copy(z_hbm.at[r + 1], row1, s_in1).wait()

        @pl.when(i > 0)
        def _():
            pltpu.make_async_copy(sv1, keys_hbm.at[r - 1], s_o1).wait()
        process_row(2 * i + 1, row1, sv1)
        pltpu.make_async_copy(sv1, keys_hbm.at[r + 1], s_o1).start()
        return 0

    lax.fori_loop(0, PAIRS, pair_body, 0)
    last = base_row + rpw
    pltpu.make_async_copy(sv0, keys_hbm.at[last - 2], s_o0).wait()
    pltpu.make_async_copy(sv1, keys_hbm.at[last - 1], s_o1).wait()
    pltpu.sync_copy(ns_v, ns_hbm.at[wid])


def _sc_compact(z, c):
    nr = z.shape[0]
    rpw = nr // NWORK
    mesh = plsc.VectorSubcoreMesh(core_axis_name="c", subcore_axis_name="s")
    kfn = pl.kernel(
        functools.partial(_sc_compact_body, rpw),
        mesh=mesh,
        compiler_params=pltpu.CompilerParams(needs_layout_passes=False),
        out_type=[
            jax.ShapeDtypeStruct((nr, SURV_CAP), jnp.float32),
            jax.ShapeDtypeStruct((NWORK, rpw * 16), jnp.int32),
        ],
        scratch_types=[
            pltpu.VMEM((D_SAE,), jnp.float32),
            pltpu.VMEM((D_SAE,), jnp.float32),
            pltpu.VMEM((SURV_CAP,), jnp.float32),
            pltpu.VMEM((SURV_CAP,), jnp.float32),
            pltpu.VMEM((rpw,), jnp.float32),
            pltpu.VMEM((rpw * 16,), jnp.int32),
            pltpu.SemaphoreType.DMA,
            pltpu.SemaphoreType.DMA,
            pltpu.SemaphoreType.DMA,
            pltpu.SemaphoreType.DMA,
        ],
    )
    return kfn(z, c)


BM = 512


def _tc_binsearch_kernel(vals_ref, ns_ref, tau_ref):
    m = jnp.int32(-2147483648)
    u = lax.bitcast_convert_type(vals_ref[...], jnp.int32)
    s = lax.shift_right_arithmetic(u, 31)
    ks = lax.bitwise_xor(u, lax.shift_right_logical(s, 1))
    col = lax.broadcasted_iota(jnp.int32, (BM, SURV_CAP), 1)
    valid = col < ns_ref[...]
    kvec = jnp.full((BM, 1), K, jnp.int32)
    prefix = jnp.zeros((BM, 1), jnp.int32)
    for b in range(32):
        bit = jnp.int32(-2147483648) if b == 0 else jnp.int32(1 << (31 - b))
        cand = lax.bitwise_or(prefix, bit)
        hits = (ks >= lax.bitwise_xor(cand, m)) & valid
        cnt = jnp.sum(hits.astype(jnp.int32), axis=1, keepdims=True)
        prefix = jnp.where(cnt >= kvec, cand, prefix)
    nki = lax.bitwise_xor(prefix, jnp.int32(-1))
    mm = lax.bitwise_or(lax.shift_right_arithmetic(nki, 31), m)
    tau_ref[...] = lax.bitcast_convert_type(
        lax.bitwise_xor(prefix, mm), jnp.float32)


def _tc_binsearch(vals, ns):
    nr = vals.shape[0]
    m_blocks = nr // BM
    return pl.pallas_call(
        _tc_binsearch_kernel,
        grid=(m_blocks,),
        in_specs=[
            pl.BlockSpec((BM, SURV_CAP), lambda j: (j, 0)),
            pl.BlockSpec((BM, 1), lambda j: (j, 0)),
        ],
        out_specs=pl.BlockSpec((BM, 1), lambda j: (j, 0)),
        out_shape=jax.ShapeDtypeStruct((nr, 1), jnp.float32),
    )(vals, ns)


def _decode_kernel(z_ref, tau_ref, w_ref, b_ref, out_ref, zs_ref):
    k = pl.program_id(1)
    z = z_ref[...]
    zs = jnp.where(z >= tau_ref[...], z, jnp.float32(0))
    zs_ref[...] = zs
    part = lax.dot_general(
        zs.astype(jnp.bfloat16), w_ref[...],
        (((1,), (1,)), ((), ())),
        preferred_element_type=jnp.float32,
    )

    @pl.when(k == 0)
    def _():
        out_ref[...] = part + b_ref[...]

    @pl.when(k != 0)
    def _():
        out_ref[...] += part


def _decode(z, tau, W_dec_bf, b_dec):
    nr = z.shape[0]
    m_blocks = nr // DM
    k_blocks = D_SAE // DK
    return pl.pallas_call(
        _decode_kernel,
        grid=(m_blocks, k_blocks),
        in_specs=[
            pl.BlockSpec((DM, DK), lambda j, k: (j, k)),
            pl.BlockSpec((DM, 1), lambda j, k: (j, 0)),
            pl.BlockSpec((D_MODEL, DK), lambda j, k: (0, k)),
            pl.BlockSpec((1, D_MODEL), lambda j, k: (0, 0)),
        ],
        out_specs=[
            pl.BlockSpec((DM, D_MODEL), lambda j, k: (j, 0)),
            pl.BlockSpec((DM, DK), lambda j, k: (j, k)),
        ],
        out_shape=[
            jax.ShapeDtypeStruct((nr, D_MODEL), jnp.float32),
            jax.ShapeDtypeStruct((nr, D_SAE), jnp.float32),
        ],
    )(z, tau, W_dec_bf, b_dec.reshape(1, D_MODEL))


NSLAB = 2


def kernel(x, W_enc, b_enc, W_dec, b_dec):
    W_enc_bf = W_enc.astype(jnp.bfloat16)
    W_dec_bf = W_dec.astype(jnp.bfloat16)
    half = N // NSLAB
    outs = []
    for si in range(NSLAB):
        xs = lax.slice_in_dim(x, si * half, (si + 1) * half, axis=0)
        z, c = _encode(xs, W_enc_bf, b_enc)
        vals, ns = _sc_compact(z, c.reshape(half))
        tau = _tc_binsearch(vals, ns.reshape(half, 16)[:, :1])
        outs.append(_decode(z, tau, W_dec_bf, b_dec))
    x_hat = jnp.concatenate([o[0] for o in outs], axis=0)
    z_sparse = jnp.concatenate([o[1] for o in outs], axis=0)
    return (x_hat, z_sparse)


# speedup vs baseline: 1.0396x; 1.0396x over previous
import functools

import jax
import jax.numpy as jnp
from jax import lax
from jax.experimental import pallas as pl
from jax.experimental.pallas import tpu as pltpu
from jax.experimental.pallas import tpu_sc as plsc

N = 8192
D_MODEL = 2048
D_SAE = 16384
K = 64

EM = 512
EN = 1024
CHUNK = 256
DM = 512
DK = 2048

NWORK = 32
ROWS_PER_W = N // NWORK
NVREG = D_SAE // 16
SURV_CAP = 2048


def _encode_kernel(x_ref, w_ref, b_ref, z_ref, c_ref, cmin_sc):
    i = pl.program_id(1)
    z = lax.dot_general(
        x_ref[...].astype(jnp.bfloat16), w_ref[...].astype(jnp.bfloat16),
        (((1,), (1,)), ((), ())),
        preferred_element_type=jnp.float32,
    )
    z = z + b_ref[...]
    z_ref[...] = z
    cm = jnp.max(z[:, 0:CHUNK], axis=1, keepdims=True)
    for k in range(1, EN // CHUNK):
        cm = jnp.minimum(cm, jnp.max(z[:, k * CHUNK:(k + 1) * CHUNK],
                                     axis=1, keepdims=True))

    @pl.when(i == 0)
    def _():
        cmin_sc[...] = cm

    @pl.when(i != 0)
    def _():
        cmin_sc[...] = jnp.minimum(cmin_sc[...], cm)

    @pl.when(i == pl.num_programs(1) - 1)
    def _():
        c_ref[...] = jnp.minimum(cmin_sc[...], cm)


def _encode(x, W_enc_bf, b_enc):
    m_blocks = N // EM
    n_blocks = D_SAE // EN
    return pl.pallas_call(
        _encode_kernel,
        grid=(m_blocks, n_blocks),
        in_specs=[
            pl.BlockSpec((EM, D_MODEL), lambda j, i: (j, 0)),
            pl.BlockSpec((EN, D_MODEL), lambda j, i: (i, 0)),
            pl.BlockSpec((1, EN), lambda j, i: (0, i)),
        ],
        out_specs=[
            pl.BlockSpec((EM, EN), lambda j, i: (j, i)),
            pl.BlockSpec((EM, 1), lambda j, i: (j, 0)),
        ],
        out_shape=[
            jax.ShapeDtypeStruct((N, D_SAE), jnp.float32),
            jax.ShapeDtypeStruct((N, 1), jnp.float32),
        ],
        scratch_shapes=[pltpu.VMEM((EM, 1), jnp.float32)],
    )(x, W_enc_bf, b_enc.reshape(1, D_SAE))


def _monotone_key(v):
    u = plsc.bitcast(v, jnp.int32)
    s = lax.shift_right_arithmetic(u, 31)
    m = lax.bitwise_or(s, jnp.int32(-2147483648))
    return plsc.bitcast(lax.bitwise_xor(u, m), jnp.uint32)


def _inv_monotone_key(k):
    ki = plsc.bitcast(k, jnp.int32)
    nki = lax.bitwise_xor(ki, jnp.int32(-1))
    m = lax.bitwise_or(
        lax.shift_right_arithmetic(nki, 31),
        jnp.int32(-2147483648))
    return plsc.bitcast(lax.bitwise_xor(ki, m), jnp.float32)


def _sc_compact_body(z_hbm, c_hbm, keys_hbm, ns_hbm,
                     row0, row1, skey, c_v, ns_v, s_in0, s_in1):
    wid = lax.axis_index("s") * 2 + lax.axis_index("c")
    base_row = wid * ROWS_PER_W

    lane = lax.iota(jnp.int32, 16)
    zero16i = jnp.zeros((16,), jnp.int32)
    cap16 = jnp.full((16,), SURV_CAP - 16, jnp.int32)

    pltpu.sync_copy(c_hbm.at[pl.ds(base_row, ROWS_PER_W)], c_v)

    def process_row(rr, row_v):
        r = base_row + rr
        cchunk = c_v[pl.ds((rr // 16) * 16, 16)]
        csel = jnp.where(lane == (rr % 16).astype(jnp.int32), cchunk,
                         jnp.float32(-jnp.inf))
        c_b = jnp.full((16,), jnp.max(csel), jnp.float32)

        def compact_body(i, off):
            v = row_v[pl.ds(i * 16, 16)]
            mask = v >= c_b
            mi = mask.astype(jnp.int32)
            dest = off + plsc.cumsum(mi) - mi
            kv = plsc.bitcast(_monotone_key(v), jnp.int32)
            plsc.store_scatter(skey, [dest], kv, mask=mask)
            pc = plsc.all_reduce_population_count(mask)
            return jnp.minimum(off + pc, cap16)
        off_vec = lax.fori_loop(0, NVREG, compact_body, zero16i)
        ns_v[pl.ds(rr * 16, 16)] = off_vec
        pltpu.sync_copy(skey, keys_hbm.at[r])
        return 0

    PAIRS = ROWS_PER_W // 2
    pltpu.make_async_copy(z_hbm.at[base_row], row0, s_in0).start()

    def pair_body(i, _):
        r = base_row + 2 * i
        pltpu.make_async_copy(z_hbm.at[r + 1], row1, s_in1).start()
        pltpu.make_async_copy(z_hbm.at[r], row0, s_in0).wait()
        process_row(2 * i, row0)

        @pl.when(i < PAIRS - 1)
        def _():
            pltpu.make_async_copy(z_hbm.at[r + 2], row0, s_in0).start()
        pltpu.make_async_copy(z_hbm.at[r + 1], row1, s_in1).wait()
        process_row(2 * i + 1, row1)
        return 0

    lax.fori_loop(0, PAIRS, pair_body, 0)
    pltpu.sync_copy(ns_v, ns_hbm.at[wid])


def _sc_compact(z, c):
    mesh = plsc.VectorSubcoreMesh(core_axis_name="c", subcore_axis_name="s")
    kfn = pl.kernel(
        _sc_compact_body,
        mesh=mesh,
        compiler_params=pltpu.CompilerParams(needs_layout_passes=False),
        out_type=[
            jax.ShapeDtypeStruct((N, SURV_CAP), jnp.int32),
            jax.ShapeDtypeStruct((NWORK, ROWS_PER_W * 16), jnp.int32),
        ],
        scratch_types=[
            pltpu.VMEM((D_SAE,), jnp.float32),
            pltpu.VMEM((D_SAE,), jnp.float32),
            pltpu.VMEM((SURV_CAP,), jnp.int32),
            pltpu.VMEM((ROWS_PER_W,), jnp.float32),
            pltpu.VMEM((ROWS_PER_W * 16,), jnp.int32),
            pltpu.SemaphoreType.DMA,
            pltpu.SemaphoreType.DMA,
        ],
    )
    return kfn(z, c)


BM = 512


def _tc_binsearch_kernel(keys_ref, ns_ref, tau_ref):
    m = jnp.int32(-2147483648)
    keys = keys_ref[...]
    ks = lax.bitwise_xor(keys, m)
    col = lax.broadcasted_iota(jnp.int32, (BM, SURV_CAP), 1)
    valid = col < ns_ref[...]
    kvec = jnp.full((BM, 1), K, jnp.int32)
    prefix = jnp.zeros((BM, 1), jnp.int32)
    for b in range(32):
        bit = jnp.int32(-2147483648) if b == 0 else jnp.int32(1 << (31 - b))
        cand = lax.bitwise_or(prefix, bit)
        hits = (ks >= lax.bitwise_xor(cand, m)) & valid
        cnt = jnp.sum(hits.astype(jnp.int32), axis=1, keepdims=True)
        prefix = jnp.where(cnt >= kvec, cand, prefix)
    nki = lax.bitwise_xor(prefix, jnp.int32(-1))
    mm = lax.bitwise_or(lax.shift_right_arithmetic(nki, 31), m)
    tau_ref[...] = lax.bitcast_convert_type(
        lax.bitwise_xor(prefix, mm), jnp.float32)


def _tc_binsearch(keys, ns):
    m_blocks = N // BM
    return pl.pallas_call(
        _tc_binsearch_kernel,
        grid=(m_blocks,),
        in_specs=[
            pl.BlockSpec((BM, SURV_CAP), lambda j: (j, 0)),
            pl.BlockSpec((BM, 1), lambda j: (j, 0)),
        ],
        out_specs=pl.BlockSpec((BM, 1), lambda j: (j, 0)),
        out_shape=jax.ShapeDtypeStruct((N, 1), jnp.float32),
    )(keys, ns)


def _decode_kernel(z_ref, tau_ref, w_ref, b_ref, out_ref, zs_ref):
    k = pl.program_id(1)
    z = z_ref[...]
    zs = jnp.where(z >= tau_ref[...], z, jnp.float32(0))
    zs_ref[...] = zs
    part = lax.dot_general(
        zs.astype(jnp.bfloat16), w_ref[...],
        (((1,), (1,)), ((), ())),
        preferred_element_type=jnp.float32,
    )

    @pl.when(k == 0)
    def _():
        out_ref[...] = part + b_ref[...]

    @pl.when(k != 0)
    def _():
        out_ref[...] += part


def _decode(z, tau, W_dec_bf, b_dec):
    m_blocks = N // DM
    k_blocks = D_SAE // DK
    return pl.pallas_call(
        _decode_kernel,
        grid=(m_blocks, k_blocks),
        in_specs=[
            pl.BlockSpec((DM, DK), lambda j, k: (j, k)),
            pl.BlockSpec((DM, 1), lambda j, k: (j, 0)),
            pl.BlockSpec((D_MODEL, DK), lambda j, k: (0, k)),
            pl.BlockSpec((1, D_MODEL), lambda j, k: (0, 0)),
        ],
        out_specs=[
            pl.BlockSpec((DM, D_MODEL), lambda j, k: (j, 0)),
            pl.BlockSpec((DM, DK), lambda j, k: (j, k)),
        ],
        out_shape=[
            jax.ShapeDtypeStruct((N, D_MODEL), jnp.float32),
            jax.ShapeDtypeStruct((N, D_SAE), jnp.float32),
        ],
    )(z, tau, W_dec_bf, b_dec.reshape(1, D_MODEL))


def kernel(x, W_enc, b_enc, W_dec, b_dec):
    z, c = _encode(x, W_enc.astype(jnp.bfloat16), b_enc)
    keys, ns = _sc_compact(z, c.reshape(N))
    tau = _tc_binsearch(keys, ns.reshape(N, 16)[:, :1])
    W_dec_bf = W_dec.astype(jnp.bfloat16)
    x_hat, z_sparse = _decode(z, tau, W_dec_bf, b_dec)
    return (x_hat, z_sparse)
